# baseline (speedup 1.0000x reference)
import functools

import jax
import jax.numpy as jnp
from jax import lax
from jax.experimental import pallas as pl
from jax.experimental.pallas import tpu as pltpu
from jax.experimental.pallas import tpu_sc as plsc

N = 10000
DH = 128
E = 320000
E2 = E + N
NC = 2
NS = 16
NW = NC * NS
K = 64
CHW = -(-E2 // (NW * K))
C0 = 146
C1 = 2 * CHW - C0
PW = CHW * K
EP = NW * PW
R = 1000
GRID = N // R

_f32 = jnp.float32
_i32 = jnp.int32


def _lin_body(x_ref, wt_ref, att_ref, h_ref, a_ref):
    h = jnp.dot(x_ref[...], wt_ref[...], preferred_element_type=_f32)
    h_ref[...] = h
    a_ref[...] = jnp.dot(h, att_ref[...], preferred_element_type=_f32)


_lin = pl.pallas_call(
    _lin_body,
    grid=(GRID,),
    in_specs=[
        pl.BlockSpec((R, DH), lambda i: (i, 0)),
        pl.BlockSpec((DH, DH), lambda i: (0, 0)),
        pl.BlockSpec((DH, 8), lambda i: (0, 0)),
    ],
    out_specs=[
        pl.BlockSpec((R, DH), lambda i: (i, 0)),
        pl.BlockSpec((R, 8), lambda i: (i, 0)),
    ],
    out_shape=[
        jax.ShapeDtypeStruct((N, DH), _f32),
        jax.ShapeDtypeStruct((N, 8), _f32),
    ],
)


def _comb_body(acc_ref, den_ref, b_ref, wt_ref, att_ref, h_ref, a_ref):
    accs = acc_ref[0] + acc_ref[1]
    den = jnp.sum(den_ref[...], axis=1)[:, None] + 1e-16
    h = accs / den + b_ref[...]
    h = jnp.maximum(h, 0.0)
    h2 = jnp.dot(h, wt_ref[...], preferred_element_type=_f32)
    h_ref[...] = h2
    a_ref[...] = jnp.dot(h2, att_ref[...], preferred_element_type=_f32)


_comb = pl.pallas_call(
    _comb_body,
    grid=(GRID,),
    in_specs=[
        pl.BlockSpec((NC, R, DH), lambda i: (0, i, 0)),
        pl.BlockSpec((R, NW), lambda i: (i, 0)),
        pl.BlockSpec((1, DH), lambda i: (0, 0)),
        pl.BlockSpec((DH, DH), lambda i: (0, 0)),
        pl.BlockSpec((DH, 8), lambda i: (0, 0)),
    ],
    out_specs=[
        pl.BlockSpec((R, DH), lambda i: (i, 0)),
        pl.BlockSpec((R, 8), lambda i: (i, 0)),
    ],
    out_shape=[
        jax.ShapeDtypeStruct((N, DH), _f32),
        jax.ShapeDtypeStruct((N, 8), _f32),
    ],
)


def _final_body(acc_ref, den_ref, b_ref, o_ref):
    accs = acc_ref[0] + acc_ref[1]
    den = jnp.sum(den_ref[...], axis=1)[:, None] + 1e-16
    o_ref[...] = accs / den + b_ref[...]


_final = pl.pallas_call(
    _final_body,
    grid=(GRID,),
    in_specs=[
        pl.BlockSpec((NC, R, DH), lambda i: (0, i, 0)),
        pl.BlockSpec((R, NW), lambda i: (i, 0)),
        pl.BlockSpec((1, DH), lambda i: (0, 0)),
    ],
    out_specs=pl.BlockSpec((R, DH), lambda i: (i, 0)),
    out_shape=jax.ShapeDtypeStruct((N, DH), _f32),
)


_sc_mesh = plsc.VectorSubcoreMesh(core_axis_name="c", subcore_axis_name="s")

_ZBLK = 40
_NBLK = N // _ZBLK


@functools.partial(
    pl.kernel,
    out_type=[
        jax.ShapeDtypeStruct((NC, N, DH), _f32),
        jax.ShapeDtypeStruct((NW * N,), _f32),
    ],
    mesh=_sc_mesh,
    compiler_params=pltpu.CompilerParams(needs_layout_passes=False),
    scratch_types=[
        pltpu.VMEM((2, K), _i32),
        pltpu.VMEM((2, K), _i32),
        pltpu.VMEM((K,), _f32),
        pltpu.VMEM((K, DH), _f32),
        pltpu.VMEM((K, DH), _f32),
        pltpu.VMEM((N,), _f32),
        pltpu.VMEM((N,), _f32),
        pltpu.VMEM((N,), _f32),
        pltpu.VMEM_SHARED((N, DH), _f32),
        pltpu.SemaphoreType.DMA,
        pltpu.SemaphoreType.DMA,
        pltpu.SemaphoreType.DMA,
        pltpu.SemaphoreType.DMA,
    ],
)
def _edge_kernel(h, asrc, adst, sdp, acc_out, den_out,
                 idx_a, idx_b, w_v, rows_a, rows_b, asrc_v, adst_v, den_v,
                 acc_sh, sem_ga, sem_gb, sem_sa, sem_sb):
    c = lax.axis_index("c")
    s = lax.axis_index("s")
    wid = c * NS + s

    pltpu.sync_copy(asrc, asrc_v)
    pltpu.sync_copy(adst, adst_v)

    z16 = jnp.zeros((16,), _f32)

    def _zden(i, carry):
        den_v[pl.ds(i * 16, 16)] = z16
        return carry

    lax.fori_loop(0, N // 16, _zden, 0)

    def _zrow(e, carry):
        for cix in range(DH // 16):
            rows_a[e, pl.ds(cix * 16, 16)] = z16
        return carry

    lax.fori_loop(0, K, _zrow, 0)
    nblk = jnp.where(s < _NBLK % NS, 1 + _NBLK // NS, _NBLK // NS)

    def _zcopy(k, carry):
        blk = k * NS + s
        pltpu.sync_copy(rows_a.at[pl.ds(0, _ZBLK)],
                        acc_sh.at[pl.ds(blk * _ZBLK, _ZBLK)])
        return carry

    lax.fori_loop(0, nblk, _zcopy, 0)
    plsc.subcore_barrier()

    def _weights(g, idx_ref):
        base = g * K
        for i in range(K // 16):
            si = idx_ref[0, pl.ds(i * 16, 16)]
            di = idx_ref[1, pl.ds(i * 16, 16)]
            z = (plsc.load_gather(asrc_v, [si])
                 + plsc.load_gather(adst_v, [di]))
            z = jnp.where(z >= 0.0, z, z * 0.2)
            wv = jnp.exp(z)
            eid = base + i * 16 + lax.iota(_i32, 16)
            wv = jnp.where(eid < E2, wv, 0.0)
            w_v[pl.ds(i * 16, 16)] = wv
            plsc.addupdate_scatter(den_v, [di], wv)

    def _scale(rows_ref):
        def _body(i, carry):
            wv16 = w_v[pl.ds(i * 16, 16)]
            for j in range(16):
                ws = wv16[j]
                e = i * 16 + j
                for cix in range(DH // 16):
                    sl = pl.ds(cix * 16, 16)
                    rows_ref[e, sl] = rows_ref[e, sl] * ws
            return carry

        lax.fori_loop(0, K // 16, _body, 0)

    cnt = jnp.where(c == 0, C0, C1)
    coff = jnp.where(c == 0, s * C0, NS * C0 + s * C1)

    def _pair(p, carry):
        g0 = coff + 2 * p
        g1 = g0 + 1
        pltpu.sync_copy(sdp.at[g0], idx_a)
        gca = pltpu.async_copy(h.at[idx_a.at[0]], rows_a, sem_ga)
        _weights(g0, idx_a)
        pltpu.sync_copy(sdp.at[g1], idx_b)
        gcb = pltpu.async_copy(h.at[idx_b.at[0]], rows_b, sem_gb)
        gca.wait()
        _scale(rows_a)
        sca = pltpu.async_copy(rows_a, acc_sh.at[idx_a.at[1]], sem_sa,
                               add=True)
        _weights(g1, idx_b)
        gcb.wait()
        _scale(rows_b)
        scb = pltpu.async_copy(rows_b, acc_sh.at[idx_b.at[1]], sem_sb,
                               add=True)
        sca.wait()
        scb.wait()
        return carry

    lax.fori_loop(0, cnt // 2, _pair, 0)
    plsc.subcore_barrier()

    pltpu.sync_copy(den_v, den_out.at[pl.ds(wid * N, N)])

    def _wcopy(k, carry):
        blk = k * NS + s
        pltpu.sync_copy(acc_sh.at[pl.ds(blk * _ZBLK, _ZBLK)],
                        acc_out.at[c, pl.ds(blk * _ZBLK, _ZBLK)])
        return carry

    lax.fori_loop(0, nblk, _wcopy, 0)


def _pad_att(att_src, att_dst):
    a = jnp.stack([att_src, att_dst], axis=1)
    return jnp.pad(a, ((0, 0), (0, 6)))


def kernel(x, edge_index, W1, att_src1, att_dst1, bias1,
           W2, att_src2, att_dst2, bias2):
    ei = edge_index.astype(_i32)
    loop = jnp.arange(N, dtype=_i32)
    pad = jnp.zeros((EP - E2,), _i32)
    srcp = jnp.concatenate([ei[0], loop, pad]).reshape(NW * CHW, K)
    dstp = jnp.concatenate([ei[1], loop, pad]).reshape(NW * CHW, K)
    sdp = jnp.stack([srcp, dstp], axis=1)

    h1, aa1 = _lin(x, W1.T, _pad_att(att_src1, att_dst1))
    acc1, den1 = _edge_kernel(h1, aa1[:, 0], aa1[:, 1], sdp)
    h2, aa2 = _comb(acc1, den1.reshape(NW, N).T, bias1.reshape(1, DH), W2.T,
                    _pad_att(att_src2, att_dst2))
    acc2, den2 = _edge_kernel(h2, aa2[:, 0], aa2[:, 1], sdp)
    return _final(acc2, den2.reshape(NW, N).T, bias2.reshape(1, DH))


# speedup vs baseline: 1.1006x; 1.1006x over previous
import functools

import jax
import jax.numpy as jnp
from jax import lax
from jax.experimental import pallas as pl
from jax.experimental.pallas import tpu as pltpu
from jax.experimental.pallas import tpu_sc as plsc

N = 10000
DH = 128
E = 320000
E2 = E + N
NC = 2
NS = 16
NW = NC * NS
K = 64
CHW = -(-E2 // (NW * K))
C0 = 178
C1 = 2 * CHW - C0
PW = CHW * K
EP = NW * PW
R = 1000
GRID = N // R

_f32 = jnp.float32
_i32 = jnp.int32


def _lin_body(x_ref, wt_ref, att_ref, h_ref, a_ref):
    h = jnp.dot(x_ref[...], wt_ref[...], preferred_element_type=_f32)
    h_ref[...] = h
    a_ref[...] = jnp.dot(h, att_ref[...], preferred_element_type=_f32)


_lin = pl.pallas_call(
    _lin_body,
    grid=(GRID,),
    in_specs=[
        pl.BlockSpec((R, DH), lambda i: (i, 0)),
        pl.BlockSpec((DH, DH), lambda i: (0, 0)),
        pl.BlockSpec((DH, 8), lambda i: (0, 0)),
    ],
    out_specs=[
        pl.BlockSpec((R, DH), lambda i: (i, 0)),
        pl.BlockSpec((R, 8), lambda i: (i, 0)),
    ],
    out_shape=[
        jax.ShapeDtypeStruct((N, DH), _f32),
        jax.ShapeDtypeStruct((N, 8), _f32),
    ],
)


def _comb_body(acc_ref, den_ref, b_ref, wt_ref, att_ref, h_ref, a_ref):
    accs = acc_ref[0] + acc_ref[1]
    den = jnp.sum(den_ref[...], axis=1)[:, None] + 1e-16
    h = accs / den + b_ref[...]
    h = jnp.maximum(h, 0.0)
    h2 = jnp.dot(h, wt_ref[...], preferred_element_type=_f32)
    h_ref[...] = h2
    a_ref[...] = jnp.dot(h2, att_ref[...], preferred_element_type=_f32)


_comb = pl.pallas_call(
    _comb_body,
    grid=(GRID,),
    in_specs=[
        pl.BlockSpec((NC, R, DH), lambda i: (0, i, 0)),
        pl.BlockSpec((R, NW), lambda i: (i, 0)),
        pl.BlockSpec((1, DH), lambda i: (0, 0)),
        pl.BlockSpec((DH, DH), lambda i: (0, 0)),
        pl.BlockSpec((DH, 8), lambda i: (0, 0)),
    ],
    out_specs=[
        pl.BlockSpec((R, DH), lambda i: (i, 0)),
        pl.BlockSpec((R, 8), lambda i: (i, 0)),
    ],
    out_shape=[
        jax.ShapeDtypeStruct((N, DH), _f32),
        jax.ShapeDtypeStruct((N, 8), _f32),
    ],
)


def _final_body(acc_ref, den_ref, b_ref, o_ref):
    accs = acc_ref[0] + acc_ref[1]
    den = jnp.sum(den_ref[...], axis=1)[:, None] + 1e-16
    o_ref[...] = accs / den + b_ref[...]


_final = pl.pallas_call(
    _final_body,
    grid=(GRID,),
    in_specs=[
        pl.BlockSpec((NC, R, DH), lambda i: (0, i, 0)),
        pl.BlockSpec((R, NW), lambda i: (i, 0)),
        pl.BlockSpec((1, DH), lambda i: (0, 0)),
    ],
    out_specs=pl.BlockSpec((R, DH), lambda i: (i, 0)),
    out_shape=jax.ShapeDtypeStruct((N, DH), _f32),
)


_sc_mesh = plsc.VectorSubcoreMesh(core_axis_name="c", subcore_axis_name="s")

_ZBLK = 40
_NBLK = N // _ZBLK


@functools.partial(
    pl.kernel,
    out_type=[
        jax.ShapeDtypeStruct((NC, N, DH), _f32),
        jax.ShapeDtypeStruct((NW * N,), _f32),
    ],
    mesh=_sc_mesh,
    compiler_params=pltpu.CompilerParams(needs_layout_passes=False),
    scratch_types=[
        pltpu.VMEM((2, K), _i32),
        pltpu.VMEM((2, K), _i32),
        pltpu.VMEM((K,), _f32),
        pltpu.VMEM((K, DH), _f32),
        pltpu.VMEM((K, DH), _f32),
        pltpu.VMEM((N,), _f32),
        pltpu.VMEM((N,), _f32),
        pltpu.VMEM((N,), _f32),
        pltpu.VMEM_SHARED((N, DH), _f32),
        pltpu.SemaphoreType.DMA,
        pltpu.SemaphoreType.DMA,
        pltpu.SemaphoreType.DMA,
        pltpu.SemaphoreType.DMA,
    ],
)
def _edge_kernel(h, asrc, adst, sdp, acc_out, den_out,
                 idx_a, idx_b, w_v, rows_a, rows_b, asrc_v, adst_v, den_v,
                 acc_sh, sem_ga, sem_gb, sem_sa, sem_sb):
    c = lax.axis_index("c")
    s = lax.axis_index("s")
    wid = c * NS + s

    pltpu.sync_copy(asrc, asrc_v)
    pltpu.sync_copy(adst, adst_v)

    z16 = jnp.zeros((16,), _f32)

    def _zden(i, carry):
        den_v[pl.ds(i * 16, 16)] = z16
        return carry

    lax.fori_loop(0, N // 16, _zden, 0)

    def _zrow(e, carry):
        for cix in range(DH // 16):
            rows_a[e, pl.ds(cix * 16, 16)] = z16
        return carry

    lax.fori_loop(0, K, _zrow, 0)
    nblk = jnp.where(s < _NBLK % NS, 1 + _NBLK // NS, _NBLK // NS)

    def _zcopy(k, carry):
        blk = k * NS + s
        pltpu.sync_copy(rows_a.at[pl.ds(0, _ZBLK)],
                        acc_sh.at[pl.ds(blk * _ZBLK, _ZBLK)])
        return carry

    lax.fori_loop(0, nblk, _zcopy, 0)
    plsc.subcore_barrier()

    def _weights(g, idx_ref):
        base = g * K
        for i in range(K // 16):
            si = idx_ref[0, pl.ds(i * 16, 16)]
            di = idx_ref[1, pl.ds(i * 16, 16)]
            z = (plsc.load_gather(asrc_v, [si])
                 + plsc.load_gather(adst_v, [di]))
            z = jnp.where(z >= 0.0, z, z * 0.2)
            wv = jnp.exp(z)
            eid = base + i * 16 + lax.iota(_i32, 16)
            wv = jnp.where(eid < E2, wv, 0.0)
            w_v[pl.ds(i * 16, 16)] = wv
            plsc.addupdate_scatter(den_v, [di], wv)

    def _scale(rows_ref):
        def _body(i, carry):
            wv16 = w_v[pl.ds(i * 16, 16)]
            for j in range(16):
                ws = wv16[j]
                e = i * 16 + j
                for cix in range(DH // 16):
                    sl = pl.ds(cix * 16, 16)
                    rows_ref[e, sl] = rows_ref[e, sl] * ws
            return carry

        lax.fori_loop(0, K // 16, _body, 0)

    cnt = jnp.where(c == 0, C0, C1)
    coff = jnp.where(c == 0, s * C0, NS * C0 + s * C1)

    def _pair(p, carry):
        g0 = coff + 2 * p
        g1 = g0 + 1
        pltpu.sync_copy(sdp.at[g0], idx_a)
        gca = pltpu.async_copy(h.at[idx_a.at[0]], rows_a, sem_ga)
        _weights(g0, idx_a)
        pltpu.sync_copy(sdp.at[g1], idx_b)
        gcb = pltpu.async_copy(h.at[idx_b.at[0]], rows_b, sem_gb)
        gca.wait()
        _scale(rows_a)
        sca = pltpu.async_copy(rows_a, acc_sh.at[idx_a.at[1]], sem_sa,
                               add=True)
        _weights(g1, idx_b)
        gcb.wait()
        _scale(rows_b)
        scb = pltpu.async_copy(rows_b, acc_sh.at[idx_b.at[1]], sem_sb,
                               add=True)
        sca.wait()
        scb.wait()
        return carry

    lax.fori_loop(0, cnt // 2, _pair, 0)
    plsc.subcore_barrier()

    pltpu.sync_copy(den_v, den_out.at[pl.ds(wid * N, N)])

    def _wcopy(k, carry):
        blk = k * NS + s
        pltpu.sync_copy(acc_sh.at[pl.ds(blk * _ZBLK, _ZBLK)],
                        acc_out.at[c, pl.ds(blk * _ZBLK, _ZBLK)])
        return carry

    lax.fori_loop(0, nblk, _wcopy, 0)


def _pad_att(att_src, att_dst):
    a = jnp.stack([att_src, att_dst], axis=1)
    return jnp.pad(a, ((0, 0), (0, 6)))


def kernel(x, edge_index, W1, att_src1, att_dst1, bias1,
           W2, att_src2, att_dst2, bias2):
    ei = edge_index.astype(_i32)
    loop = jnp.arange(N, dtype=_i32)
    pad = jnp.zeros((EP - E2,), _i32)
    srcp = jnp.concatenate([ei[0], loop, pad]).reshape(NW * CHW, K)
    dstp = jnp.concatenate([ei[1], loop, pad]).reshape(NW * CHW, K)
    sdp = jnp.stack([srcp, dstp], axis=1)

    h1, aa1 = _lin(x, W1.T, _pad_att(att_src1, att_dst1))
    acc1, den1 = _edge_kernel(h1, aa1[:, 0], aa1[:, 1], sdp)
    h2, aa2 = _comb(acc1, den1.reshape(NW, N).T, bias1.reshape(1, DH), W2.T,
                    _pad_att(att_src2, att_dst2))
    acc2, den2 = _edge_kernel(h2, aa2[:, 0], aa2[:, 1], sdp)
    return _final(acc2, den2.reshape(NW, N).T, bias2.reshape(1, DH))
